# baseline (speedup 1.0000x reference)
import functools

import jax
import jax.numpy as jnp
from jax.experimental import pallas as pl
from jax.experimental.pallas import tpu as pltpu

_TB = 1024


def _res_kernel(x_ref, w_ref, o_ref, sw_ref, mnmx_ref, cval_ref, coh_ref,
                *, tb, bpb, k):
    i = pl.program_id(0)

    @pl.when(i == 0)
    def _init():
        w0 = w_ref[...]
        sw_ref[...] = jax.nn.softmax(w0, axis=1)
        mnmx_ref[0:1, :] = jnp.min(w0, axis=0, keepdims=True)
        mnmx_ref[1:2, :] = jnp.max(w0, axis=0, keepdims=True)
        cval_ref[...] = jnp.zeros_like(cval_ref)
        coh_ref[...] = jnp.zeros_like(coh_ref)

    x = x_ref[...]
    logits = jnp.dot(x, sw_ref[...], preferred_element_type=jnp.float32)

    val = jnp.max(logits, axis=1, keepdims=True)
    oh = (logits == val).astype(jnp.float32)

    row0 = jax.lax.broadcasted_iota(jnp.int32, (tb, 1), 0) == 0
    v = jnp.where(row0, cval_ref[...], jnp.roll(val, 1, axis=0))
    onehot = jnp.where(row0, coh_ref[...], jnp.roll(oh, 1, axis=0))

    cval_ref[...] = val[tb - 1:tb, :]
    coh_ref[...] = oh[tb - 1:tb, :]

    rows = jax.lax.dot_general(
        onehot, w_ref[...],
        dimension_numbers=(((1,), (1,)), ((), ())),
        preferred_element_type=jnp.float32)

    cmn = jnp.sum(onehot * mnmx_ref[0:1, :], axis=1, keepdims=True)
    cmx = jnp.sum(onehot * mnmx_ref[1:2, :], axis=1, keepdims=True)
    pos = v >= 0.0
    mn_w = jnp.where(pos, v * cmn, v * cmx)
    mx_w = jnp.where(pos, v * cmx, v * cmn)
    inv = 1.0 / (mx_w - mn_w)
    a = v * inv
    c = 1.0 - mn_w * inv
    zero_row = row0 & (i % bpb == 0)
    a = jnp.where(zero_row, 0.0, a)
    c = jnp.where(zero_row, 1.0, c)
    o_ref[...] = x * (rows * a + c)


def kernel(input, w):
    b, t, d = input.shape
    k = w.shape[1]
    n = b * t
    tb = _TB
    bpb = t // tb
    xf = input.reshape(n, d)
    out = pl.pallas_call(
        functools.partial(_res_kernel, tb=tb, bpb=bpb, k=k),
        grid=(n // tb,),
        in_specs=[
            pl.BlockSpec((tb, d), lambda i: (i, 0)),
            pl.BlockSpec((d, k), lambda i: (0, 0)),
        ],
        out_specs=pl.BlockSpec((tb, d), lambda i: (i, 0)),
        out_shape=jax.ShapeDtypeStruct((n, d), jnp.float32),
        scratch_shapes=[
            pltpu.VMEM((d, k), jnp.float32),
            pltpu.VMEM((2, k), jnp.float32),
            pltpu.VMEM((1, 1), jnp.float32),
            pltpu.VMEM((1, k), jnp.float32),
        ],
    )(xf, w)
    return out.reshape(b, t, d)
